# Initial kernel scaffold; baseline (speedup 1.0000x reference)
import jax
import jax.numpy as jnp
from jax.experimental import pallas as pl


def kernel(x, edge_index, W1, b1, W2, b2, W3, b3, Wfc, bfc):
    raise NotImplementedError("write your pallas kernel here")


# speedup vs baseline: 8.3189x; 8.3189x over previous
import functools

import jax
import jax.numpy as jnp
from jax import lax
from jax.experimental import pallas as pl
from jax.experimental.pallas import tpu as pltpu
from jax.experimental.pallas import tpu_sc as plsc

N = 10000
E = 320000
NP = 10240
NC = 2
NS = 16
NW = NC * NS
B = 128
NB = (E + NW * B - 1) // (NW * B)
EPW = NB * B
EP = NW * EPW
RPT = NP // NS

_mesh = functools.partial(
    plsc.VectorSubcoreMesh, core_axis_name="c", subcore_axis_name="s")


def _deg_body(dst_hbm, ones_hbm, z_hbm, out_hbm, dst_v, ones_v, acc):
    c = lax.axis_index("c")
    s = lax.axis_index("s")
    w = c * NS + s
    pltpu.sync_copy(dst_hbm.at[w], dst_v)
    pltpu.sync_copy(ones_hbm, ones_v)
    pltpu.sync_copy(z_hbm, acc.at[pl.ds(s * RPT, RPT)])
    plsc.subcore_barrier()
    def step(b, carry):
        pltpu.sync_copy(ones_v, acc.at[dst_v.at[b]], add=True)
        return carry
    lax.fori_loop(0, NB, step, 0)
    plsc.subcore_barrier()
    pltpu.sync_copy(acc.at[pl.ds(s * RPT, RPT)],
                    out_hbm.at[c, pl.ds(s * RPT, RPT)])


def _sc_degree(dst3, ones128, z128):
    return pl.kernel(
        _deg_body,
        out_type=jax.ShapeDtypeStruct((NC, NP, 128), jnp.float32),
        mesh=_mesh(),
        scratch_types=[
            pltpu.VMEM((NB, B), jnp.int32),
            pltpu.VMEM((B, 128), jnp.float32),
            pltpu.VMEM_SHARED((NP, 128), jnp.float32),
        ],
    )(dst3, ones128, z128)


def _agg_body(nslabs, *refs):
    xs = refs[:nslabs]
    src_hbm, dst_hbm, z_hbm = refs[nslabs:nslabs + 3]
    outs = refs[nslabs + 3:2 * nslabs + 3]
    src_v, dst_v, rows_v, acc, sem = refs[2 * nslabs + 3:]
    c = lax.axis_index("c")
    s = lax.axis_index("s")
    w = c * NS + s
    pltpu.sync_copy(src_hbm.at[w], src_v)
    pltpu.sync_copy(dst_hbm.at[w], dst_v)
    for j in range(nslabs):
        pltpu.sync_copy(z_hbm, acc.at[pl.ds(s * RPT, RPT)])
        plsc.subcore_barrier()
        def step(b, carry, j=j):
            pltpu.async_copy(xs[j].at[src_v.at[b]], rows_v, sem).wait()
            pltpu.sync_copy(rows_v, acc.at[dst_v.at[b]], add=True)
            return carry
        lax.fori_loop(0, NB, step, 0)
        plsc.subcore_barrier()
        pltpu.sync_copy(acc.at[pl.ds(s * RPT, RPT)],
                        outs[j].at[c, pl.ds(s * RPT, RPT)])
        plsc.subcore_barrier()


def _sc_aggregate(slabs, src3, dst3, z128):
    k = len(slabs)
    return pl.kernel(
        functools.partial(_agg_body, k),
        out_type=[jax.ShapeDtypeStruct((NC, NP, 128), jnp.float32)] * k,
        mesh=_mesh(),
        scratch_types=[
            pltpu.VMEM((NB, B), jnp.int32),
            pltpu.VMEM((NB, B), jnp.int32),
            pltpu.VMEM((B, 128), jnp.float32),
            pltpu.VMEM_SHARED((NP, 128), jnp.float32),
            pltpu.SemaphoreType.DMA,
        ],
    )(*slabs, src3, dst3, z128)


def _prep_body(degp, x, xs_o, dinv_o):
    deg = degp[0] + degp[1] + 1.0
    dinv = lax.rsqrt(deg)[:, 0:1]
    xs_o[...] = x[...] * dinv
    dinv_o[...] = jnp.broadcast_to(dinv, dinv_o.shape)


def _tc_prep(degp, xp, rb=512):
    grid = (NP // rb,)
    return pl.pallas_call(
        _prep_body,
        grid=grid,
        in_specs=[
            pl.BlockSpec((NC, rb, 128), lambda i: (0, i, 0)),
            pl.BlockSpec((rb, 128), lambda i: (i, 0)),
        ],
        out_specs=[
            pl.BlockSpec((rb, 128), lambda i: (i, 0)),
            pl.BlockSpec((rb, 128), lambda i: (i, 0)),
        ],
        out_shape=[
            jax.ShapeDtypeStruct((NP, 128), jnp.float32),
            jax.ShapeDtypeStruct((NP, 128), jnp.float32),
        ],
    )(degp, xp)


def _l12_body(s1p, xs, dinvb, w1, b1, w2, o0, o1, o2, o3):
    dinv = dinvb[...][:, 0:1]
    agg0 = dinvb[...] * (s1p[0] + s1p[1] + xs[...])
    h1 = jnp.maximum(jnp.dot(agg0, w1[...],
                             preferred_element_type=jnp.float32) + b1[...], 0.0)
    t2 = jnp.dot(h1, w2[...], preferred_element_type=jnp.float32) * dinv
    o0[...] = t2[:, 0:128]
    o1[...] = t2[:, 128:256]
    o2[...] = t2[:, 256:384]
    o3[...] = t2[:, 384:512]


def _tc_layers12(s1p, xs, dinvb, w1, b1r, w2, rb=256):
    grid = (NP // rb,)
    return pl.pallas_call(
        _l12_body,
        grid=grid,
        in_specs=[
            pl.BlockSpec((NC, rb, 128), lambda i: (0, i, 0)),
            pl.BlockSpec((rb, 128), lambda i: (i, 0)),
            pl.BlockSpec((rb, 128), lambda i: (i, 0)),
            pl.BlockSpec((128, 1024), lambda i: (0, 0)),
            pl.BlockSpec((1, 1024), lambda i: (0, 0)),
            pl.BlockSpec((1024, 512), lambda i: (0, 0)),
        ],
        out_specs=[pl.BlockSpec((rb, 128), lambda i: (i, 0))] * 4,
        out_shape=[jax.ShapeDtypeStruct((NP, 128), jnp.float32)] * 4,
    )(s1p, xs, dinvb, w1, b1r, w2)


def _l3_body(p0, p1, p2, p3, t0, t1, t2, t3, dinvb, b2, w3, o):
    dinv = dinvb[...][:, 0:1]
    cols = [p0[0] + p0[1] + t0[...], p1[0] + p1[1] + t1[...],
            p2[0] + p2[1] + t2[...], p3[0] + p3[1] + t3[...]]
    s2 = jnp.concatenate(cols, axis=1)
    h2 = jnp.maximum(dinv * s2 + b2[...], 0.0)
    o[...] = jnp.dot(h2, w3[...], preferred_element_type=jnp.float32) * dinv


def _tc_layer3(s2ps, t2s, dinvb, b2r, w3, rb=256):
    grid = (NP // rb,)
    return pl.pallas_call(
        _l3_body,
        grid=grid,
        in_specs=(
            [pl.BlockSpec((NC, rb, 128), lambda i: (0, i, 0))] * 4
            + [pl.BlockSpec((rb, 128), lambda i: (i, 0))] * 4
            + [
                pl.BlockSpec((rb, 128), lambda i: (i, 0)),
                pl.BlockSpec((1, 512), lambda i: (0, 0)),
                pl.BlockSpec((512, 128), lambda i: (0, 0)),
            ]
        ),
        out_specs=pl.BlockSpec((rb, 128), lambda i: (i, 0)),
        out_shape=jax.ShapeDtypeStruct((NP, 128), jnp.float32),
    )(*s2ps, *t2s, dinvb, b2r, w3)


def _l4_body(s3p, t3s, dinvb, b3, wfc, bfc, o):
    dinv = dinvb[...][:, 0:1]
    h3 = jnp.maximum(dinv * (s3p[0] + s3p[1] + t3s[...]) + b3[...], 0.0)
    z = jnp.dot(h3, wfc[...], preferred_element_type=jnp.float32) + bfc[...]
    m = jnp.max(z, axis=1, keepdims=True)
    lse = m + jnp.log(jnp.sum(jnp.exp(z - m), axis=1, keepdims=True))
    o[...] = z - lse


def _tc_layer4(s3p, t3s, dinvb, b3r, wfc, bfcr, rb=256):
    grid = (NP // rb,)
    return pl.pallas_call(
        _l4_body,
        grid=grid,
        in_specs=[
            pl.BlockSpec((NC, rb, 128), lambda i: (0, i, 0)),
            pl.BlockSpec((rb, 128), lambda i: (i, 0)),
            pl.BlockSpec((rb, 128), lambda i: (i, 0)),
            pl.BlockSpec((1, 128), lambda i: (0, 0)),
            pl.BlockSpec((128, 64), lambda i: (0, 0)),
            pl.BlockSpec((1, 64), lambda i: (0, 0)),
        ],
        out_specs=pl.BlockSpec((rb, 64), lambda i: (i, 0)),
        out_shape=jax.ShapeDtypeStruct((NP, 64), jnp.float32),
    )(s3p, t3s, dinvb, b3r, wfc, bfcr)


def kernel(x, edge_index, W1, b1, W2, b2, W3, b3, Wfc, bfc):
    xp = jnp.pad(x, ((0, NP - N), (0, 0)))
    pad = jnp.full((EP - E,), N, dtype=jnp.int32)
    src3 = jnp.concatenate([edge_index[0], pad]).reshape(NW, NB, B)
    dst3 = jnp.concatenate([edge_index[1], pad]).reshape(NW, NB, B)
    z128 = jnp.zeros((RPT, 128), jnp.float32)
    ones128 = jnp.ones((B, 128), jnp.float32)
    b1r = b1.reshape(1, 1024)
    b2r = b2.reshape(1, 512)
    b3r = b3.reshape(1, 128)
    bfcr = bfc.reshape(1, 64)

    degp = _sc_degree(dst3, ones128, z128)
    xs, dinvb = _tc_prep(degp, xp)
    (s1p,) = _sc_aggregate([xs], src3, dst3, z128)
    t2s = _tc_layers12(s1p, xs, dinvb, W1, b1r, W2)
    s2ps = _sc_aggregate(list(t2s), src3, dst3, z128)
    t3s = _tc_layer3(s2ps, t2s, dinvb, b2r, W3)
    (s3p,) = _sc_aggregate([t3s], src3, dst3, z128)
    out = _tc_layer4(s3p, t3s, dinvb, b3r, Wfc, bfcr)
    return out[:N]
